# baseline (speedup 1.0000x reference)
import functools

import jax
import jax.numpy as jnp
from jax import lax
from jax.experimental import pallas as pl
from jax.experimental.pallas import tpu as pltpu
from jax.experimental.pallas import tpu_sc as plsc

_INF = 1 << 20


def _shift_right(x, s, fill):
    pad = jnp.full((x.shape[0], s), fill, x.dtype)
    return jnp.concatenate([pad, x[:, : x.shape[1] - s]], axis=1)


def _levels_tc_kernel(num_levels, tok_ref, emb_ref, lev_ref, semb_ref):
    tok = tok_ref[...]
    is_open = (tok == 40) | (tok == 91) | (tok == 123)
    is_close = (tok == 41) | (tok == 93) | (tok == 125)

    f = jnp.float32
    a = jnp.where(is_open, f(1), f(0)) - jnp.where(is_close, f(1), f(0))
    lo = jnp.where(is_close, f(0), f(-_INF))
    hi = jnp.where(is_open, f(num_levels - 1), f(_INF))

    s = 1
    seq = tok.shape[1]
    while s < seq:
        pa = _shift_right(a, s, 0.0)
        plo = _shift_right(lo, s, float(-_INF))
        phi = _shift_right(hi, s, float(_INF))
        na = pa + a
        nlo = jnp.clip(plo + a, lo, hi)
        nhi = jnp.clip(phi + a, lo, hi)
        a, lo, hi = na, nlo, nhi
        s *= 2

    lev_ref[...] = jnp.clip(a, lo, hi).astype(jnp.int32)
    semb_ref[...] = emb_ref[...] * 0.15


@functools.lru_cache(maxsize=None)
def _make_sc_lookup(n_rows, num_levels, d, chunk, nbuf):
    info = plsc.get_sparse_core_info()
    nw = info.num_cores * info.num_subcores
    lanes = info.num_lanes
    rows_per_w = n_rows // nw
    n_chunks = rows_per_w // chunk
    assert n_chunks % nbuf == 0 and d % lanes == 0
    mesh = plsc.VectorSubcoreMesh(core_axis_name="c", subcore_axis_name="s")

    @functools.partial(
        pl.kernel,
        mesh=mesh,
        out_type=jax.ShapeDtypeStruct((n_rows, d), jnp.float32),
        scratch_types=[
            pltpu.VMEM((num_levels * d,), jnp.float32),
            pltpu.VMEM((rows_per_w + lanes,), jnp.int32),
            *([pltpu.VMEM((chunk, d), jnp.float32)] * nbuf),
            pltpu.SemaphoreType.DMA,
            *([pltpu.SemaphoreType.DMA] * nbuf),
        ],
    )
    def lookup(idx_hbm, table_hbm, out_hbm, table_v, idx_v, *scratch):
        bufs = scratch[:nbuf]
        lsem = scratch[nbuf]
        ssem = scratch[nbuf + 1 :]
        wid = lax.axis_index("s") * info.num_cores + lax.axis_index("c")
        base = wid * rows_per_w

        idx_dst = idx_v.at[pl.ds(0, rows_per_w)]
        pltpu.async_copy(idx_hbm.at[pl.ds(base, rows_per_w)], idx_dst, lsem)
        pltpu.async_copy(table_hbm, table_v, lsem)
        pltpu.make_async_copy(idx_hbm.at[pl.ds(base, rows_per_w)], idx_dst, lsem).wait()
        pltpu.make_async_copy(table_hbm, table_v, lsem).wait()

        def fill(c, b):
            buf = bufs[b]

            def row(r, carry):
                lvl = idx_v[pl.ds(c * chunk + r, lanes)][0]
                src = lvl * d
                dst = buf.at[r]
                for j in range(0, d, lanes):
                    dst[pl.ds(j, lanes)] = table_v[pl.ds(src + j, lanes)]
                return carry

            lax.fori_loop(0, chunk, row, 0)

        def start_scatter(c, b):
            pltpu.async_copy(
                bufs[b], out_hbm.at[pl.ds(base + c * chunk, chunk)], ssem[b]
            )

        def wait_scatter(c, b):
            pltpu.make_async_copy(
                bufs[b], out_hbm.at[pl.ds(base + c * chunk, chunk)], ssem[b]
            ).wait()

        for b in range(nbuf):
            fill(b, b)
            start_scatter(b, b)

        def outer(c0, carry):
            for b in range(nbuf):
                c = c0 * nbuf + b
                wait_scatter(c - nbuf, b)
                fill(c, b)
                start_scatter(c, b)
            return carry

        lax.fori_loop(1, n_chunks // nbuf, outer, 0)
        for b in range(nbuf):
            wait_scatter(n_chunks - nbuf + b, b)

    return lookup


def kernel(token_ids, classifications, level_emb):
    del classifications
    b, s = token_ids.shape
    num_levels, d = level_emb.shape

    levels, scaled_emb = pl.pallas_call(
        functools.partial(_levels_tc_kernel, num_levels),
        out_shape=[
            jax.ShapeDtypeStruct((b, s), jnp.int32),
            jax.ShapeDtypeStruct((num_levels, d), level_emb.dtype),
        ],
    )(token_ids, level_emb)

    idx = levels.reshape(b * s)
    out = _make_sc_lookup(b * s, num_levels, d, 32, 2)(idx, scaled_emb.reshape(-1))
    return out.reshape(b, s, d)


# speedup vs baseline: 3.7067x; 3.7067x over previous
import functools

import jax
import jax.numpy as jnp
from jax import lax
from jax.experimental import pallas as pl
from jax.experimental.pallas import tpu as pltpu
from jax.experimental.pallas import tpu_sc as plsc

_INF = 1 << 20


def _shift_right(x, s, fill):
    pad = jnp.full((x.shape[0], s), fill, x.dtype)
    return jnp.concatenate([pad, x[:, : x.shape[1] - s]], axis=1)


def _levels_tc_kernel(num_levels, tok_ref, emb_ref, lev_ref, semb_ref):
    tok = tok_ref[...]
    is_open = (tok == 40) | (tok == 91) | (tok == 123)
    is_close = (tok == 41) | (tok == 93) | (tok == 125)

    f = jnp.float32
    a = jnp.where(is_open, f(1), f(0)) - jnp.where(is_close, f(1), f(0))
    lo = jnp.where(is_close, f(0), f(-_INF))
    hi = jnp.where(is_open, f(num_levels - 1), f(_INF))

    s = 1
    seq = tok.shape[1]
    while s < seq:
        pa = _shift_right(a, s, 0.0)
        plo = _shift_right(lo, s, float(-_INF))
        phi = _shift_right(hi, s, float(_INF))
        na = pa + a
        nlo = jnp.clip(plo + a, lo, hi)
        nhi = jnp.clip(phi + a, lo, hi)
        a, lo, hi = na, nlo, nhi
        s *= 2

    lev_ref[...] = jnp.clip(a, lo, hi).astype(jnp.int32)
    semb_ref[...] = emb_ref[...] * 0.15


@functools.lru_cache(maxsize=None)
def _make_sc_lookup(n_rows, num_levels, d, unroll):
    info = plsc.get_sparse_core_info()
    nw = info.num_cores * info.num_subcores
    lanes = info.num_lanes
    rows_per_w = n_rows // nw
    assert rows_per_w % unroll == 0 and d % lanes == 0
    mesh = plsc.VectorSubcoreMesh(core_axis_name="c", subcore_axis_name="s")

    @functools.partial(
        pl.kernel,
        mesh=mesh,
        out_type=jax.ShapeDtypeStruct((n_rows, d), jnp.float32),
        scratch_types=[
            pltpu.VMEM((num_levels, d), jnp.float32),
            pltpu.VMEM((rows_per_w + lanes,), jnp.int32),
            pltpu.SemaphoreType.DMA,
            pltpu.SemaphoreType.DMA,
        ],
    )
    def lookup(idx_hbm, table_hbm, out_hbm, table_v, idx_v, lsem, ssem):
        wid = lax.axis_index("s") * info.num_cores + lax.axis_index("c")
        base = wid * rows_per_w

        idx_dst = idx_v.at[pl.ds(0, rows_per_w)]
        pltpu.async_copy(idx_hbm.at[pl.ds(base, rows_per_w)], idx_dst, lsem)
        pltpu.async_copy(table_hbm, table_v, lsem)
        pltpu.make_async_copy(idx_hbm.at[pl.ds(base, rows_per_w)], idx_dst, lsem).wait()
        pltpu.make_async_copy(table_hbm, table_v, lsem).wait()

        def issue(r0, carry):
            for u in range(unroll):
                r = r0 * unroll + u
                lvl = idx_v[pl.ds(r, lanes)][0]
                pltpu.async_copy(
                    table_v.at[pl.ds(lvl, 1)],
                    out_hbm.at[pl.ds(base + r, 1)],
                    ssem,
                )
            return carry

        lax.fori_loop(0, rows_per_w // unroll, issue, 0)

        def drain(r, carry):
            pltpu.make_async_copy(
                table_v.at[pl.ds(0, 1)], out_hbm.at[pl.ds(base, 1)], ssem
            ).wait()
            return carry

        lax.fori_loop(0, rows_per_w, drain, 0)

    return lookup


def kernel(token_ids, classifications, level_emb):
    del classifications
    b, s = token_ids.shape
    num_levels, d = level_emb.shape

    levels, scaled_emb = pl.pallas_call(
        functools.partial(_levels_tc_kernel, num_levels),
        out_shape=[
            jax.ShapeDtypeStruct((b, s), jnp.int32),
            jax.ShapeDtypeStruct((num_levels, d), level_emb.dtype),
        ],
    )(token_ids, level_emb)

    idx = levels.reshape(b * s)
    out = _make_sc_lookup(b * s, num_levels, d, 8)(idx, scaled_emb)
    return out.reshape(b, s, d)


# speedup vs baseline: 3.8001x; 1.0252x over previous
import functools

import jax
import jax.numpy as jnp
from jax import lax
from jax.experimental import pallas as pl
from jax.experimental.pallas import tpu as pltpu
from jax.experimental.pallas import tpu_sc as plsc

_INF = 1 << 20


def _shift_right(x, s, fill):
    pad = jnp.full((x.shape[0], s), fill, x.dtype)
    return jnp.concatenate([pad, x[:, : x.shape[1] - s]], axis=1)


def _levels_tc_kernel(num_levels, tok_ref, emb_ref, lev_ref, semb_ref):
    tok = tok_ref[...]
    is_open = (tok == 40) | (tok == 91) | (tok == 123)
    is_close = (tok == 41) | (tok == 93) | (tok == 125)

    f = jnp.float32
    a = jnp.where(is_open, f(1), f(0)) - jnp.where(is_close, f(1), f(0))
    lo = jnp.where(is_close, f(0), f(-_INF))
    hi = jnp.where(is_open, f(num_levels - 1), f(_INF))

    s = 1
    seq = tok.shape[1]
    while s < seq:
        pa = _shift_right(a, s, 0.0)
        plo = _shift_right(lo, s, float(-_INF))
        phi = _shift_right(hi, s, float(_INF))
        na = pa + a
        nlo = jnp.clip(plo + a, lo, hi)
        nhi = jnp.clip(phi + a, lo, hi)
        a, lo, hi = na, nlo, nhi
        s *= 2

    lev_ref[...] = jnp.clip(a, lo, hi).astype(jnp.int32)

    scaled = emb_ref[...] * 0.15
    for w in range(semb_ref.shape[0]):
        semb_ref[w] = scaled


@functools.lru_cache(maxsize=None)
def _make_sc_lookup(n_rows, num_levels, d, unroll):
    info = plsc.get_sparse_core_info()
    nw = info.num_cores * info.num_subcores
    lanes = info.num_lanes
    rows_per_w = n_rows // nw
    assert rows_per_w % unroll == 0 and d % lanes == 0
    mesh = plsc.VectorSubcoreMesh(core_axis_name="c", subcore_axis_name="s")

    @functools.partial(
        pl.kernel,
        mesh=mesh,
        out_type=jax.ShapeDtypeStruct((n_rows, d), jnp.float32),
        scratch_types=[
            pltpu.VMEM((num_levels, d), jnp.float32),
            pltpu.VMEM((rows_per_w + lanes,), jnp.int32),
            pltpu.SemaphoreType.DMA,
            pltpu.SemaphoreType.DMA,
        ],
    )
    def lookup(idx_hbm, table_hbm, out_hbm, table_v, idx_v, lsem, ssem):
        wid = lax.axis_index("s") * info.num_cores + lax.axis_index("c")
        base = wid * rows_per_w

        idx_dst = idx_v.at[pl.ds(0, rows_per_w)]
        tab_src = table_hbm.at[pl.ds(wid * num_levels, num_levels)]
        pltpu.async_copy(idx_hbm.at[pl.ds(base, rows_per_w)], idx_dst, lsem)
        pltpu.async_copy(tab_src, table_v, lsem)
        pltpu.make_async_copy(idx_hbm.at[pl.ds(base, rows_per_w)], idx_dst, lsem).wait()
        pltpu.make_async_copy(tab_src, table_v, lsem).wait()

        def issue(r0, carry):
            for u in range(unroll):
                r = r0 * unroll + u
                lvl = idx_v[pl.ds(r, lanes)][0]
                pltpu.async_copy(
                    table_v.at[pl.ds(lvl, 1)],
                    out_hbm.at[pl.ds(base + r, 1)],
                    ssem,
                )
            return carry

        lax.fori_loop(0, rows_per_w // unroll, issue, 0)

        def drain(r, carry):
            pltpu.make_async_copy(
                table_v.at[pl.ds(0, 1)], out_hbm.at[pl.ds(base, 1)], ssem
            ).wait()
            return carry

        lax.fori_loop(0, rows_per_w, drain, 0)

    return lookup


def kernel(token_ids, classifications, level_emb):
    del classifications
    b, s = token_ids.shape
    num_levels, d = level_emb.shape

    info = plsc.get_sparse_core_info()
    nw = info.num_cores * info.num_subcores

    levels, scaled_emb = pl.pallas_call(
        functools.partial(_levels_tc_kernel, num_levels),
        out_shape=[
            jax.ShapeDtypeStruct((b, s), jnp.int32),
            jax.ShapeDtypeStruct((nw, num_levels, d), level_emb.dtype),
        ],
    )(token_ids, level_emb)

    idx = levels.reshape(b * s)
    out = _make_sc_lookup(b * s, num_levels, d, 8)(
        idx, scaled_emb.reshape(nw * num_levels, d)
    )
    return out.reshape(b, s, d)
